# Initial kernel scaffold; baseline (speedup 1.0000x reference)
import jax
import jax.numpy as jnp
from jax import lax
from jax.experimental import pallas as pl
from jax.experimental.pallas import tpu as pltpu
from jax.experimental.pallas import tpu_sc as plsc

T, E, D = 8192, 16, 1024
CAP = 1280
NC, NS = 2, 16
NW = NC * NS
TPW = T // NW

_MESH = plsc.VectorSubcoreMesh(core_axis_name="c", subcore_axis_name="s")


def _route_body(logits_hbm, gv_hbm, hist_hbm, logits_v, gv_v, hist_v):
    wid = lax.axis_index("s") * NC + lax.axis_index("c")
    base = wid * TPW
    pltpu.sync_copy(logits_hbm.at[pl.ds(base, TPW)], logits_v)
    iota = lax.broadcasted_iota(jnp.int32, (E,), 0)

    def body(t, c):
        v = logits_v[t]
        m1 = jnp.max(v)
        e0 = jnp.min(jnp.where(v == m1, iota, E))
        oh0 = iota == e0
        vm = jnp.where(oh0, -jnp.inf, v)
        m2 = jnp.max(vm)
        e1 = jnp.min(jnp.where(vm == m2, iota, E))
        oh1 = iota == e1
        q = jnp.exp(jnp.broadcast_to(m2 - m1, (E,)))
        g1 = q / (1.0 + q)
        g0 = 1.0 - g1
        gv_v[t] = jnp.where(oh0, g0, jnp.where(oh1, g1, 0.0))
        return c + oh0.astype(jnp.int32) + oh1.astype(jnp.int32)

    c = lax.fori_loop(0, TPW, body, jnp.zeros((E,), jnp.int32))
    hist_v[...] = c
    pltpu.sync_copy(gv_v, gv_hbm.at[pl.ds(base, TPW)])
    pltpu.sync_copy(hist_v, hist_hbm.at[wid])


_route = pl.kernel(
    _route_body,
    out_type=(
        jax.ShapeDtypeStruct((T, E), jnp.float32),
        jax.ShapeDtypeStruct((NW, E), jnp.int32),
    ),
    mesh=_MESH,
    scratch_types=[
        pltpu.VMEM((TPW, E), jnp.float32),
        pltpu.VMEM((TPW, E), jnp.float32),
        pltpu.VMEM((E,), jnp.int32),
    ],
)


def _scale_body(gv_hbm, hist_hbm, scale_hbm, loads_hbm,
                gv_v, hist_v, scale_v, loads_v):
    wid = lax.axis_index("s") * NC + lax.axis_index("c")
    base = wid * TPW
    pltpu.sync_copy(hist_hbm, hist_v)
    pltpu.sync_copy(gv_hbm.at[pl.ds(base, TPW)], gv_v)

    zeros = jnp.zeros((E,), jnp.int32)

    def obody(w, acc):
        off, tot = acc
        row = hist_v[w]
        return (off + row * (w < wid).astype(jnp.int32), tot + row)

    off, tot = lax.fori_loop(0, NW, obody, (zeros, zeros))

    def tbody(t, c):
        gvt = gv_v[t]
        keep = c < CAP
        scale_v[t] = jnp.sum(jnp.where(keep, gvt, 0.0))
        return c + (gvt > 0.0).astype(jnp.int32)

    lax.fori_loop(0, TPW, tbody, off)
    pltpu.sync_copy(scale_v, scale_hbm.at[pl.ds(base, TPW)])

    @pl.when(wid == 0)
    def _():
        loads_v[...] = jnp.minimum(tot, CAP).astype(jnp.float32)
        pltpu.sync_copy(loads_v, loads_hbm)


_scale = pl.kernel(
    _scale_body,
    out_type=(
        jax.ShapeDtypeStruct((T,), jnp.float32),
        jax.ShapeDtypeStruct((E,), jnp.float32),
    ),
    mesh=_MESH,
    scratch_types=[
        pltpu.VMEM((TPW, E), jnp.float32),
        pltpu.VMEM((NW, E), jnp.int32),
        pltpu.VMEM((TPW,), jnp.float32),
        pltpu.VMEM((E,), jnp.float32),
    ],
)


def _tc_body(x_ref, s_ref, o_ref):
    o_ref[...] = x_ref[...] * s_ref[...]


_BT = 256


def _tc_scale(x, scale2d):
    return pl.pallas_call(
        _tc_body,
        grid=(T // _BT,),
        in_specs=[
            pl.BlockSpec((_BT, D), lambda i: (i, 0)),
            pl.BlockSpec((_BT, 1), lambda i: (i, 0)),
        ],
        out_specs=pl.BlockSpec((_BT, D), lambda i: (i, 0)),
        out_shape=jax.ShapeDtypeStruct((T, D), jnp.float32),
    )(x, scale2d)


@jax.jit
def kernel(x, router_logits):
    gv, hist = _route(router_logits)
    scale, loads = _scale(gv, hist)
    y = _tc_scale(x, scale.reshape(T, 1))
    return y, loads


# speedup vs baseline: 11.1952x; 11.1952x over previous
import jax
import jax.numpy as jnp
from jax import lax
from jax.experimental import pallas as pl
from jax.experimental.pallas import tpu as pltpu
from jax.experimental.pallas import tpu_sc as plsc

T, E, D = 8192, 16, 1024
CAP = 1280
NC, NS = 2, 16
NW = NC * NS
TPW = T // NW

_MESH = plsc.VectorSubcoreMesh(core_axis_name="c", subcore_axis_name="s")


def _route_body(logits_hbm, gv_hbm, hist_hbm, logits_v, gv_v, hist_v):
    wid = lax.axis_index("s") * NC + lax.axis_index("c")
    base = wid * TPW
    pltpu.sync_copy(logits_hbm.at[pl.ds(base, TPW)], logits_v)
    iota = lax.broadcasted_iota(jnp.int32, (E,), 0)

    def body(t, c):
        v = logits_v[t]
        m1 = jnp.max(v)
        e0 = jnp.min(jnp.where(v == m1, iota, E))
        oh0 = iota == e0
        vm = jnp.where(oh0, -jnp.inf, v)
        m2 = jnp.max(vm)
        e1 = jnp.min(jnp.where(vm == m2, iota, E))
        oh1 = iota == e1
        q = jnp.exp(jnp.broadcast_to(m2 - m1, (E,)))
        g1 = q / (1.0 + q)
        g0 = 1.0 - g1
        gv_v[t] = jnp.where(oh0, g0, jnp.where(oh1, g1, 0.0))
        return c + oh0.astype(jnp.int32) + oh1.astype(jnp.int32)

    c = lax.fori_loop(0, TPW, body, jnp.zeros((E,), jnp.int32))
    hist_v[...] = c
    pltpu.sync_copy(gv_v, gv_hbm.at[pl.ds(base, TPW)])
    pltpu.sync_copy(hist_v, hist_hbm.at[wid])


_route = pl.kernel(
    _route_body,
    out_type=(
        jax.ShapeDtypeStruct((T, E), jnp.float32),
        jax.ShapeDtypeStruct((NW, E), jnp.int32),
    ),
    mesh=_MESH,
    scratch_types=[
        pltpu.VMEM((TPW, E), jnp.float32),
        pltpu.VMEM((TPW, E), jnp.float32),
        pltpu.VMEM((E,), jnp.int32),
    ],
    compiler_params=pltpu.CompilerParams(needs_layout_passes=False),
)


def _scale_body(gv_hbm, hist_hbm, scale_hbm, loads_hbm,
                gv_v, hist_v, scale_v, loads_v):
    wid = lax.axis_index("s") * NC + lax.axis_index("c")
    base = wid * TPW
    pltpu.sync_copy(hist_hbm, hist_v)
    pltpu.sync_copy(gv_hbm.at[pl.ds(base, TPW)], gv_v)

    zeros = jnp.zeros((E,), jnp.int32)

    def obody(w, acc):
        off, tot = acc
        row = hist_v[w]
        return (off + row * (w < wid).astype(jnp.int32), tot + row)

    off, tot = lax.fori_loop(0, NW, obody, (zeros, zeros))
    iota = lax.broadcasted_iota(jnp.int32, (E,), 0)

    def gbody(g, c):
        sacc = jnp.zeros((E,), jnp.float32)
        for j in range(16):
            gvt = gv_v[g * 16 + j]
            keep = c < CAP
            s = jnp.sum(jnp.where(keep, gvt, 0.0))
            sacc = jnp.where(iota == j, s, sacc)
            c = c + (gvt > 0.0).astype(jnp.int32)
        scale_v[pl.ds(g * 16, 16)] = sacc
        return c

    lax.fori_loop(0, TPW // 16, gbody, off)
    pltpu.sync_copy(scale_v, scale_hbm.at[pl.ds(base, TPW)])

    @pl.when(wid == 0)
    def _():
        loads_v[...] = jnp.minimum(tot, CAP).astype(jnp.float32)
        pltpu.sync_copy(loads_v, loads_hbm)


_scale = pl.kernel(
    _scale_body,
    out_type=(
        jax.ShapeDtypeStruct((T,), jnp.float32),
        jax.ShapeDtypeStruct((E,), jnp.float32),
    ),
    mesh=_MESH,
    scratch_types=[
        pltpu.VMEM((TPW, E), jnp.float32),
        pltpu.VMEM((NW, E), jnp.int32),
        pltpu.VMEM((TPW,), jnp.float32),
        pltpu.VMEM((E,), jnp.float32),
    ],
    compiler_params=pltpu.CompilerParams(needs_layout_passes=False),
)


def _tc_body(x_ref, s_ref, o_ref):
    o_ref[...] = x_ref[...] * s_ref[...]


_BT = 256


def _tc_scale(x, scale2d):
    return pl.pallas_call(
        _tc_body,
        grid=(T // _BT,),
        in_specs=[
            pl.BlockSpec((_BT, D), lambda i: (i, 0)),
            pl.BlockSpec((_BT, 1), lambda i: (i, 0)),
        ],
        out_specs=pl.BlockSpec((_BT, D), lambda i: (i, 0)),
        out_shape=jax.ShapeDtypeStruct((T, D), jnp.float32),
    )(x, scale2d)


@jax.jit
def kernel(x, router_logits):
    gv, hist = _route(router_logits)
    scale, loads = _scale(gv, hist)
    y = _tc_scale(x, scale.reshape(T, 1))
    return y, loads


# speedup vs baseline: 15.3038x; 1.3670x over previous
import jax
import jax.numpy as jnp
from jax import lax
from jax.experimental import pallas as pl
from jax.experimental.pallas import tpu as pltpu
from jax.experimental.pallas import tpu_sc as plsc

T, E, D = 8192, 16, 1024
CAP = 1280
NC, NS = 2, 16
NW = NC * NS
TPW = T // NW

_MESH = plsc.VectorSubcoreMesh(core_axis_name="c", subcore_axis_name="s")


def _route_body(logits_hbm, gv_hbm, logits_v, gv_v):
    wid = lax.axis_index("s") * NC + lax.axis_index("c")
    base = wid * TPW
    pltpu.sync_copy(logits_hbm.at[pl.ds(base, TPW)], logits_v)
    iota = lax.broadcasted_iota(jnp.int32, (E,), 0)

    def body(t, carry):
        v = logits_v[t]
        m1 = jnp.max(v)
        e0 = jnp.min(jnp.where(v == m1, iota, E))
        oh0 = iota == e0
        vm = jnp.where(oh0, -jnp.inf, v)
        m2 = jnp.max(vm)
        e1 = jnp.min(jnp.where(vm == m2, iota, E))
        oh1 = iota == e1
        q = jnp.exp(jnp.broadcast_to(m2 - m1, (E,)))
        g1 = q / (1.0 + q)
        g0 = 1.0 - g1
        gv_v[t] = jnp.where(oh0, g0, jnp.where(oh1, g1, 0.0))
        return carry

    lax.fori_loop(0, TPW, body, 0)
    pltpu.sync_copy(gv_v, gv_hbm.at[pl.ds(base, TPW)])


_route = pl.kernel(
    _route_body,
    out_type=jax.ShapeDtypeStruct((T, E), jnp.float32),
    mesh=_MESH,
    scratch_types=[
        pltpu.VMEM((TPW, E), jnp.float32),
        pltpu.VMEM((TPW, E), jnp.float32),
    ],
    compiler_params=pltpu.CompilerParams(needs_layout_passes=False),
)


_BT = 512
_NB = T // _BT


def _tc_body(x_ref, gv_ref, tril_ref, y_ref, loads_ref, cnt_ref):
    i = pl.program_id(0)

    @pl.when(i == 0)
    def _():
        cnt_ref[...] = jnp.zeros_like(cnt_ref)

    gv = gv_ref[...]
    oh = (gv > 0.0).astype(jnp.float32)
    cum = jnp.dot(tril_ref[...], oh, preferred_element_type=jnp.float32)
    c = cum + cnt_ref[...]
    keep = c < float(CAP)
    scale = jnp.sum(jnp.where(keep, gv, 0.0), axis=1, keepdims=True)
    y_ref[...] = x_ref[...] * scale
    cnt_ref[...] = cnt_ref[...] + jnp.sum(oh, axis=0, keepdims=True)

    @pl.when(i == _NB - 1)
    def _():
        loads_ref[...] = jnp.minimum(cnt_ref[...], float(CAP))


def _tc_scan_scale(x, gv, tril):
    return pl.pallas_call(
        _tc_body,
        grid=(_NB,),
        in_specs=[
            pl.BlockSpec((_BT, D), lambda i: (i, 0)),
            pl.BlockSpec((_BT, E), lambda i: (i, 0)),
            pl.BlockSpec((_BT, _BT), lambda i: (0, 0)),
        ],
        out_specs=[
            pl.BlockSpec((_BT, D), lambda i: (i, 0)),
            pl.BlockSpec((1, E), lambda i: (0, 0)),
        ],
        out_shape=[
            jax.ShapeDtypeStruct((T, D), jnp.float32),
            jax.ShapeDtypeStruct((1, E), jnp.float32),
        ],
        scratch_shapes=[pltpu.VMEM((1, E), jnp.float32)],
    )(x, gv, tril)


@jax.jit
def kernel(x, router_logits):
    gv = _route(router_logits)
    row = lax.broadcasted_iota(jnp.int32, (_BT, _BT), 0)
    col = lax.broadcasted_iota(jnp.int32, (_BT, _BT), 1)
    tril = (col < row).astype(jnp.float32)
    y, loads = _tc_scan_scale(x, gv, tril)
    return y, loads.reshape(E)


# speedup vs baseline: 15.9781x; 1.0441x over previous
import numpy as np

import jax
import jax.numpy as jnp
from jax import lax
from jax.experimental import pallas as pl
from jax.experimental.pallas import tpu as pltpu
from jax.experimental.pallas import tpu_sc as plsc

T, E, D = 8192, 16, 1024
CAP = 1280
NC, NS = 2, 16
NW = NC * NS
TPW = T // NW
L = 16

_MESH = plsc.VectorSubcoreMesh(core_axis_name="c", subcore_axis_name="s")


def _route_body(logits_hbm, gv_hbm, logits_v, gv_v):
    wid = lax.axis_index("s") * NC + lax.axis_index("c")
    base = wid * TPW
    pltpu.sync_copy(logits_hbm.at[:, pl.ds(base, TPW)], logits_v)

    def body(g, carry):
        toff = g * L
        v = [logits_v[e, pl.ds(toff, L)] for e in range(E)]
        m1 = v[0]
        e0 = jnp.zeros((L,), jnp.int32)
        m2 = jnp.full((L,), -jnp.inf, jnp.float32)
        e1 = jnp.zeros((L,), jnp.int32)
        for e in range(1, E):
            upd1 = v[e] > m1
            upd2 = jnp.logical_and(jnp.logical_not(upd1), v[e] > m2)
            m2 = jnp.where(upd1, m1, jnp.where(upd2, v[e], m2))
            e1 = jnp.where(upd1, e0, jnp.where(upd2, e, e1))
            m1 = jnp.where(upd1, v[e], m1)
            e0 = jnp.where(upd1, e, e0)
        q = jnp.exp(m2 - m1)
        g1 = q / (1.0 + q)
        g0 = 1.0 - g1
        for e in range(E):
            gv_v[e, pl.ds(toff, L)] = jnp.where(
                e0 == e, g0, jnp.where(e1 == e, g1, 0.0))
        return carry

    lax.fori_loop(0, TPW // L, body, 0)
    pltpu.sync_copy(gv_v, gv_hbm.at[:, pl.ds(base, TPW)])


_route = pl.kernel(
    _route_body,
    out_type=jax.ShapeDtypeStruct((E, T), jnp.float32),
    mesh=_MESH,
    scratch_types=[
        pltpu.VMEM((E, TPW), jnp.float32),
        pltpu.VMEM((E, TPW), jnp.float32),
    ],
    compiler_params=pltpu.CompilerParams(needs_layout_passes=False),
)


_BT = 512
_NB = T // _BT

_TRIU = np.triu(np.ones((_BT, _BT), np.float32), 1)


def _tc_body(x_ref, gv_ref, triu_ref, y_ref, loads_ref, cnt_ref):
    i = pl.program_id(0)

    @pl.when(i == 0)
    def _():
        cnt_ref[...] = jnp.zeros_like(cnt_ref)

    gv = gv_ref[...]
    oh = (gv > 0.0).astype(jnp.float32)
    cum = jnp.dot(oh, triu_ref[...], preferred_element_type=jnp.float32)
    c = cum + cnt_ref[...]
    gvk = jnp.where(c < float(CAP), gv, 0.0)
    scale = lax.dot_general(
        gvk, jnp.full((E, 1), 1.0, jnp.float32),
        dimension_numbers=(((0,), (0,)), ((), ())),
        preferred_element_type=jnp.float32,
    )
    y_ref[...] = x_ref[...] * scale
    cnt_ref[...] = cnt_ref[...] + jnp.sum(oh, axis=1, keepdims=True)

    @pl.when(i == _NB - 1)
    def _():
        loads_ref[...] = jnp.minimum(cnt_ref[...], float(CAP))


def _tc_scan_scale(x, gv_t, triu):
    return pl.pallas_call(
        _tc_body,
        grid=(_NB,),
        in_specs=[
            pl.BlockSpec((_BT, D), lambda i: (i, 0)),
            pl.BlockSpec((E, _BT), lambda i: (0, i)),
            pl.BlockSpec((_BT, _BT), lambda i: (0, 0)),
        ],
        out_specs=[
            pl.BlockSpec((_BT, D), lambda i: (i, 0)),
            pl.BlockSpec((E, 1), lambda i: (0, 0)),
        ],
        out_shape=[
            jax.ShapeDtypeStruct((T, D), jnp.float32),
            jax.ShapeDtypeStruct((E, 1), jnp.float32),
        ],
        scratch_shapes=[pltpu.VMEM((E, 1), jnp.float32)],
    )(x, gv_t, triu)


@jax.jit
def kernel(x, router_logits):
    gv_t = _route(router_logits.T)
    y, loads = _tc_scan_scale(x, gv_t, jnp.asarray(_TRIU))
    return y, loads.reshape(E)
